# initial kernel scaffold (unmeasured)
import jax
import jax.numpy as jnp
from jax import lax
from jax.experimental import pallas as pl
from jax.experimental.pallas import tpu as pltpu

B, H, D, BS = 8, 8, 64, 16
NB = 64
PAGES_LOCAL = 64
KEYS = PAGES_LOCAL * BS
NEG = -1e30
SCALE = D ** -0.5


def kernel(Q, K, V, bt, lens):
    Q2 = Q.reshape(B, H * D)
    K2 = K.reshape(KEYS, H * D)
    V2 = V.reshape(KEYS, H * D)
    lens2 = lens.reshape(B, 1)

    def body(q_ref, k_ref, v_ref, bt_ref, lens_ref, out_ref,
             o_send, m_send, l_send, o_recv, m_recv, l_recv,
             send_sems, recv_sems):
        my_x = lax.axis_index("x")
        my_y = lax.axis_index("y")
        partner = (my_x, 1 - my_y)

        barrier = pltpu.get_barrier_semaphore()
        pl.semaphore_signal(barrier, inc=1, device_id=partner,
                            device_id_type=pl.DeviceIdType.MESH)
        pl.semaphore_wait(barrier, 1)

        jj = lax.broadcasted_iota(jnp.int32, (B, NB), 1)
        valid = jj < lens_ref[...]
        local_page = bt_ref[...] - my_y * PAGES_LOCAL
        key_page = lax.broadcasted_iota(jnp.int32, (B, KEYS), 1) // BS
        hits = (local_page[:, :, None] == key_page[:, None, :]) \
            & valid[:, :, None]
        w = jnp.sum(hits.astype(jnp.float32), axis=1)
        has_key = w > 0.0

        for h in range(H):
            q_h = q_ref[:, h * D:(h + 1) * D]
            k_h = k_ref[:, h * D:(h + 1) * D]
            v_h = v_ref[:, h * D:(h + 1) * D]
            s = lax.dot_general(q_h, k_h, (((1,), (1,)), ((), ())),
                                preferred_element_type=jnp.float32) * SCALE
            s = jnp.where(has_key, s, NEG)
            m_h = jnp.max(s, axis=1, keepdims=True)
            p = jnp.exp(s - m_h) * w
            l_h = jnp.sum(p, axis=1, keepdims=True)
            o_h = lax.dot_general(p, v_h, (((1,), (0,)), ((), ())),
                                  preferred_element_type=jnp.float32)
            o_send[:, h * D:(h + 1) * D] = o_h
            m_send[:, h:h + 1] = m_h
            l_send[:, h:h + 1] = l_h

        rdmas = []
        for i, (src, dst) in enumerate(
            ((o_send, o_recv), (m_send, m_recv), (l_send, l_recv))
        ):
            r = pltpu.make_async_remote_copy(
                src_ref=src, dst_ref=dst,
                send_sem=send_sems.at[i], recv_sem=recv_sems.at[i],
                device_id=partner, device_id_type=pl.DeviceIdType.MESH)
            r.start()
            rdmas.append(r)
        for r in rdmas:
            r.wait()

        for h in range(H):
            m1 = m_send[:, h:h + 1]
            m2 = m_recv[:, h:h + 1]
            mn = jnp.maximum(m1, m2)
            a1 = jnp.exp(m1 - mn)
            a2 = jnp.exp(m2 - mn)
            lsum = a1 * l_send[:, h:h + 1] + a2 * l_recv[:, h:h + 1]
            o = a1 * o_send[:, h * D:(h + 1) * D] \
                + a2 * o_recv[:, h * D:(h + 1) * D]
            out_ref[:, 0, h, :] = o / lsum

    return pl.pallas_call(
        body,
        out_shape=jax.ShapeDtypeStruct((B, 1, H, D), jnp.float32),
        in_specs=[pl.BlockSpec(memory_space=pltpu.VMEM)] * 5,
        out_specs=pl.BlockSpec(memory_space=pltpu.VMEM),
        scratch_shapes=[
            pltpu.VMEM((B, H * D), jnp.float32),
            pltpu.VMEM((B, H), jnp.float32),
            pltpu.VMEM((B, H), jnp.float32),
            pltpu.VMEM((B, H * D), jnp.float32),
            pltpu.VMEM((B, H), jnp.float32),
            pltpu.VMEM((B, H), jnp.float32),
            pltpu.SemaphoreType.DMA((3,)),
            pltpu.SemaphoreType.DMA((3,)),
        ],
        compiler_params=pltpu.CompilerParams(collective_id=0),
    )(Q2, K2, V2, bt, lens2)


# baseline (device time: 14372 ns/iter reference)
import jax
import jax.numpy as jnp
from jax import lax
from jax.experimental import pallas as pl
from jax.experimental.pallas import tpu as pltpu

B, H, D, BS = 8, 8, 64, 16
NB = 64
PAGES_LOCAL = 64
KEYS = PAGES_LOCAL * BS
NEG = -1e30
SCALE = D ** -0.5


def kernel(Q, K, V, bt, lens):
    Q2 = Q.reshape(B, H * D)
    K2 = K.reshape(KEYS, H * D)
    V2 = V.reshape(KEYS, H * D)
    lens2 = lens.reshape(B, 1)

    def body(q_ref, k_ref, v_ref, bt_ref, lens_ref, out_ref,
             o_send, m_send, l_send, o_recv, m_recv, l_recv,
             send_sems, recv_sems):
        my_x = lax.axis_index("x")
        my_y = lax.axis_index("y")
        partner = (my_x, 1 - my_y)

        barrier = pltpu.get_barrier_semaphore()
        pl.semaphore_signal(barrier, inc=1, device_id=partner,
                            device_id_type=pl.DeviceIdType.MESH)
        pl.semaphore_wait(barrier, 1)

        lens_v = lens_ref[...]
        local_page = bt_ref[...] - my_y * PAGES_LOCAL
        key_page = lax.broadcasted_iota(jnp.int32, (B, KEYS), 1) // BS
        w = jnp.zeros((B, KEYS), jnp.float32)
        for j in range(NB):
            pj = local_page[:, j:j + 1]
            vj = j < lens_v
            w = w + jnp.where((key_page == pj) & vj, 1.0, 0.0)
        has_key = w > 0.0

        for h in range(H):
            q_h = q_ref[:, h * D:(h + 1) * D]
            k_h = k_ref[:, h * D:(h + 1) * D]
            v_h = v_ref[:, h * D:(h + 1) * D]
            s = lax.dot_general(q_h, k_h, (((1,), (1,)), ((), ())),
                                preferred_element_type=jnp.float32) * SCALE
            s = jnp.where(has_key, s, NEG)
            m_h = jnp.max(s, axis=1, keepdims=True)
            p = jnp.exp(s - m_h) * w
            l_h = jnp.sum(p, axis=1, keepdims=True)
            o_h = lax.dot_general(p, v_h, (((1,), (0,)), ((), ())),
                                  preferred_element_type=jnp.float32)
            o_send[:, h * D:(h + 1) * D] = o_h
            m_send[:, h:h + 1] = m_h
            l_send[:, h:h + 1] = l_h

        rdmas = []
        for i, (src, dst) in enumerate(
            ((o_send, o_recv), (m_send, m_recv), (l_send, l_recv))
        ):
            r = pltpu.make_async_remote_copy(
                src_ref=src, dst_ref=dst,
                send_sem=send_sems.at[i], recv_sem=recv_sems.at[i],
                device_id=partner, device_id_type=pl.DeviceIdType.MESH)
            r.start()
            rdmas.append(r)
        for r in rdmas:
            r.wait()

        for h in range(H):
            m1 = m_send[:, h:h + 1]
            m2 = m_recv[:, h:h + 1]
            mn = jnp.maximum(m1, m2)
            a1 = jnp.exp(m1 - mn)
            a2 = jnp.exp(m2 - mn)
            lsum = a1 * l_send[:, h:h + 1] + a2 * l_recv[:, h:h + 1]
            o = a1 * o_send[:, h * D:(h + 1) * D] \
                + a2 * o_recv[:, h * D:(h + 1) * D]
            out_ref[:, 0, h, :] = o / lsum

    return pl.pallas_call(
        body,
        out_shape=jax.ShapeDtypeStruct((B, 1, H, D), jnp.float32),
        in_specs=[pl.BlockSpec(memory_space=pltpu.VMEM)] * 5,
        out_specs=pl.BlockSpec(memory_space=pltpu.VMEM),
        scratch_shapes=[
            pltpu.VMEM((B, H * D), jnp.float32),
            pltpu.VMEM((B, H), jnp.float32),
            pltpu.VMEM((B, H), jnp.float32),
            pltpu.VMEM((B, H * D), jnp.float32),
            pltpu.VMEM((B, H), jnp.float32),
            pltpu.VMEM((B, H), jnp.float32),
            pltpu.SemaphoreType.DMA((3,)),
            pltpu.SemaphoreType.DMA((3,)),
        ],
        compiler_params=pltpu.CompilerParams(collective_id=0),
    )(Q2, K2, V2, bt, lens2)


# device time: 14047 ns/iter; 1.0231x vs baseline; 1.0231x over previous
import jax
import jax.numpy as jnp
from jax import lax
from jax.experimental import pallas as pl
from jax.experimental.pallas import tpu as pltpu

B, H, D, BS = 8, 8, 64, 16
NB = 64
PAGES_LOCAL = 64
KEYS = PAGES_LOCAL * BS
NEG = -1e30
SCALE = D ** -0.5


def kernel(Q, K, V, bt, lens):
    Q2 = Q.reshape(B, H * D)
    K2 = K.reshape(KEYS, H * D)
    V2 = V.reshape(KEYS, H * D)
    lens2 = lens.reshape(B, 1)

    def body(q_ref, k_ref, v_ref, bt_ref, lens_ref, out_ref,
             o_send, m_send, l_send, o_recv, m_recv, l_recv,
             send_sems, recv_sems):
        my_x = lax.axis_index("x")
        my_y = lax.axis_index("y")
        partner = (my_x, 1 - my_y)

        barrier = pltpu.get_barrier_semaphore()
        pl.semaphore_signal(barrier, inc=1, device_id=partner,
                            device_id_type=pl.DeviceIdType.MESH)
        pl.semaphore_wait(barrier, 1)

        lens_v = lens_ref[...]
        local_page = bt_ref[...] - my_y * PAGES_LOCAL
        key_page = lax.broadcasted_iota(jnp.int32, (B, KEYS), 1) // BS
        w = jnp.zeros((B, KEYS), jnp.float32)
        for j in range(NB):
            pj = local_page[:, j:j + 1]
            vj = j < lens_v
            w = w + jnp.where((key_page == pj) & vj, 1.0, 0.0)

        qt = jnp.concatenate([q_ref[...]] * H, axis=0)
        sub = lax.broadcasted_iota(jnp.int32, (H * B, H * D), 0)
        lane = lax.broadcasted_iota(jnp.int32, (H * B, H * D), 1)
        qex = jnp.where((lane // D) == (sub // B), qt, 0.0)

        s = lax.dot_general(qex, k_ref[...], (((1,), (1,)), ((), ())),
                            preferred_element_type=jnp.float32) * SCALE
        w_t = jnp.concatenate([w] * H, axis=0)
        s = jnp.where(w_t > 0.0, s, NEG)
        m_all = jnp.max(s, axis=1, keepdims=True)
        p = jnp.exp(s - m_all) * w_t
        l_all = jnp.sum(p, axis=1, keepdims=True)

        m_send[...] = m_all
        l_send[...] = l_all
        rdma_m = pltpu.make_async_remote_copy(
            src_ref=m_send, dst_ref=m_recv,
            send_sem=send_sems.at[1], recv_sem=recv_sems.at[1],
            device_id=partner, device_id_type=pl.DeviceIdType.MESH)
        rdma_m.start()
        rdma_l = pltpu.make_async_remote_copy(
            src_ref=l_send, dst_ref=l_recv,
            send_sem=send_sems.at[2], recv_sem=recv_sems.at[2],
            device_id=partner, device_id_type=pl.DeviceIdType.MESH)
        rdma_l.start()

        o_all = lax.dot_general(p, v_ref[...], (((1,), (0,)), ((), ())),
                                preferred_element_type=jnp.float32)
        o_send[...] = o_all
        rdma_o = pltpu.make_async_remote_copy(
            src_ref=o_send, dst_ref=o_recv,
            send_sem=send_sems.at[0], recv_sem=recv_sems.at[0],
            device_id=partner, device_id_type=pl.DeviceIdType.MESH)
        rdma_o.start()

        rdma_m.wait()
        rdma_l.wait()
        m2 = m_recv[...]
        mn = jnp.maximum(m_all, m2)
        a1 = jnp.exp(m_all - mn)
        a2 = jnp.exp(m2 - mn)
        lsum = a1 * l_all + a2 * l_recv[...]

        rdma_o.wait()
        o = (a1 * o_all + a2 * o_recv[...]) / lsum

        for h in range(H):
            out_ref[:, 0, h, :] = o[h * B:(h + 1) * B, h * D:(h + 1) * D]

    return pl.pallas_call(
        body,
        out_shape=jax.ShapeDtypeStruct((B, 1, H, D), jnp.float32),
        in_specs=[pl.BlockSpec(memory_space=pltpu.VMEM)] * 5,
        out_specs=pl.BlockSpec(memory_space=pltpu.VMEM),
        scratch_shapes=[
            pltpu.VMEM((H * B, H * D), jnp.float32),
            pltpu.VMEM((H * B, 1), jnp.float32),
            pltpu.VMEM((H * B, 1), jnp.float32),
            pltpu.VMEM((H * B, H * D), jnp.float32),
            pltpu.VMEM((H * B, 1), jnp.float32),
            pltpu.VMEM((H * B, 1), jnp.float32),
            pltpu.SemaphoreType.DMA((3,)),
            pltpu.SemaphoreType.DMA((3,)),
        ],
        compiler_params=pltpu.CompilerParams(collective_id=0),
    )(Q2, K2, V2, bt, lens2)


# device time: 12370 ns/iter; 1.1618x vs baseline; 1.1356x over previous
import jax
import jax.numpy as jnp
from jax import lax
from jax.experimental import pallas as pl
from jax.experimental.pallas import tpu as pltpu

B, H, D, BS = 8, 8, 64, 16
NB = 64
PAGES_LOCAL = 64
KEYS = PAGES_LOCAL * BS
SCALE = D ** -0.5


def kernel(Q, K, V, bt, lens):
    Q2 = Q.reshape(B, H * D)
    K2 = K.reshape(KEYS, H * D)
    V2 = V.reshape(KEYS, H * D)
    lens2 = lens.reshape(B, 1)

    def body(q_ref, k_ref, v_ref, bt_ref, lens_ref, out_ref,
             o_send, l_send, o_recv, l_recv, send_sems, recv_sems):
        my_x = lax.axis_index("x")
        my_y = lax.axis_index("y")
        partner = (my_x, 1 - my_y)

        barrier = pltpu.get_barrier_semaphore()
        pl.semaphore_signal(barrier, inc=1, device_id=partner,
                            device_id_type=pl.DeviceIdType.MESH)
        pl.semaphore_wait(barrier, 1)

        lens_v = lens_ref[...]
        local_page = bt_ref[...] - my_y * PAGES_LOCAL
        key_page = lax.broadcasted_iota(jnp.int32, (B, KEYS), 1) // BS
        w = jnp.zeros((B, KEYS), jnp.float32)
        for j in range(NB):
            pj = local_page[:, j:j + 1]
            vj = j < lens_v
            w = w + jnp.where((key_page == pj) & vj, 1.0, 0.0)

        qt = jnp.concatenate([q_ref[...]] * H, axis=0)
        sub = lax.broadcasted_iota(jnp.int32, (H * B, H * D), 0)
        lane = lax.broadcasted_iota(jnp.int32, (H * B, H * D), 1)
        qex = jnp.where((lane // D) == (sub // B), qt, 0.0)

        s = lax.dot_general(qex, k_ref[...], (((1,), (1,)), ((), ())),
                            preferred_element_type=jnp.float32) * SCALE
        w_t = jnp.concatenate([w] * H, axis=0)
        p = jnp.exp(s) * w_t
        l_all = jnp.sum(p, axis=1, keepdims=True)

        l_send[...] = l_all
        rdma_l = pltpu.make_async_remote_copy(
            src_ref=l_send, dst_ref=l_recv,
            send_sem=send_sems.at[1], recv_sem=recv_sems.at[1],
            device_id=partner, device_id_type=pl.DeviceIdType.MESH)
        rdma_l.start()

        o_all = lax.dot_general(p, v_ref[...], (((1,), (0,)), ((), ())),
                                preferred_element_type=jnp.float32)
        for h in range(H):
            o_send[:, h * D:(h + 1) * D] = \
                o_all[h * B:(h + 1) * B, h * D:(h + 1) * D]
        rdma_o = pltpu.make_async_remote_copy(
            src_ref=o_send, dst_ref=o_recv,
            send_sem=send_sems.at[0], recv_sem=recv_sems.at[0],
            device_id=partner, device_id_type=pl.DeviceIdType.MESH)
        rdma_o.start()

        rdma_l.wait()
        lsum = l_all + l_recv[...]
        rdma_o.wait()
        o = o_send[...] + o_recv[...]
        for h in range(H):
            out_ref[:, 0, h, :] = \
                o[:, h * D:(h + 1) * D] / lsum[h * B:(h + 1) * B, :]

    return pl.pallas_call(
        body,
        out_shape=jax.ShapeDtypeStruct((B, 1, H, D), jnp.float32),
        in_specs=[pl.BlockSpec(memory_space=pltpu.VMEM)] * 5,
        out_specs=pl.BlockSpec(memory_space=pltpu.VMEM),
        scratch_shapes=[
            pltpu.VMEM((B, H * D), jnp.float32),
            pltpu.VMEM((H * B, 1), jnp.float32),
            pltpu.VMEM((B, H * D), jnp.float32),
            pltpu.VMEM((H * B, 1), jnp.float32),
            pltpu.SemaphoreType.DMA((2,)),
            pltpu.SemaphoreType.DMA((2,)),
        ],
        compiler_params=pltpu.CompilerParams(collective_id=0),
    )(Q2, K2, V2, bt, lens2)


# device time: 12206 ns/iter; 1.1775x vs baseline; 1.0134x over previous
import jax
import jax.numpy as jnp
from jax import lax
from jax.experimental import pallas as pl
from jax.experimental.pallas import tpu as pltpu

B, H, D, BS = 8, 8, 64, 16
NB = 64
PAGES_LOCAL = 64
KEYS = PAGES_LOCAL * BS
SCALE = D ** -0.5
MSG = H * D + H


def kernel(Q, K, V, bt, lens):
    Q2 = Q.reshape(B, H * D)
    K2 = K.reshape(KEYS, H * D)
    V2 = V.reshape(KEYS, H * D)
    lens2 = lens.reshape(B, 1)

    def body(q_ref, k_ref, v_ref, bt_ref, lens_ref, out_ref,
             msg_send, msg_recv, send_sem, recv_sem):
        my_x = lax.axis_index("x")
        my_y = lax.axis_index("y")
        partner = (my_x, 1 - my_y)

        barrier = pltpu.get_barrier_semaphore()
        pl.semaphore_signal(barrier, inc=1, device_id=partner,
                            device_id_type=pl.DeviceIdType.MESH)

        lens_v = lens_ref[...]
        local_page = bt_ref[...] - my_y * PAGES_LOCAL
        page_iota = lax.broadcasted_iota(jnp.int32, (B, PAGES_LOCAL), 1)
        cnt = jnp.zeros((B, PAGES_LOCAL), jnp.float32)
        for j in range(NB):
            pj = local_page[:, j:j + 1]
            vj = j < lens_v
            cnt = cnt + jnp.where((page_iota == pj) & vj, 1.0, 0.0)
        expand = jnp.where(
            lax.broadcasted_iota(jnp.int32, (PAGES_LOCAL, KEYS), 1) // BS
            == lax.broadcasted_iota(jnp.int32, (PAGES_LOCAL, KEYS), 0),
            1.0, 0.0)
        w = lax.dot_general(cnt, expand, (((1,), (0,)), ((), ())),
                            preferred_element_type=jnp.float32)

        qt = jnp.concatenate([q_ref[...]] * H, axis=0)
        sub = lax.broadcasted_iota(jnp.int32, (H * B, H * D), 0)
        lane = lax.broadcasted_iota(jnp.int32, (H * B, H * D), 1)
        qex = jnp.where((lane // D) == (sub // B), qt * SCALE, 0.0)

        s = lax.dot_general(qex, k_ref[...], (((1,), (1,)), ((), ())),
                            preferred_element_type=jnp.float32)
        w_t = jnp.concatenate([w] * H, axis=0)
        p = jnp.exp(s) * w_t
        l_all = jnp.sum(p, axis=1, keepdims=True)
        for h in range(H):
            msg_send[:, H * D + h:H * D + h + 1] = l_all[h * B:(h + 1) * B, :]

        o_all = lax.dot_general(p, v_ref[...], (((1,), (0,)), ((), ())),
                                preferred_element_type=jnp.float32)
        for h in range(H):
            msg_send[:, h * D:(h + 1) * D] = \
                o_all[h * B:(h + 1) * B, h * D:(h + 1) * D]

        pl.semaphore_wait(barrier, 1)

        rdma = pltpu.make_async_remote_copy(
            src_ref=msg_send, dst_ref=msg_recv,
            send_sem=send_sem, recv_sem=recv_sem,
            device_id=partner, device_id_type=pl.DeviceIdType.MESH)
        rdma.start()
        rdma.wait()

        lsum = msg_send[:, H * D:] + msg_recv[:, H * D:]
        inv_l = 1.0 / lsum
        o = msg_send[:, :H * D] + msg_recv[:, :H * D]
        for h in range(H):
            out_ref[:, h * D:(h + 1) * D] = \
                o[:, h * D:(h + 1) * D] * inv_l[:, h:h + 1]

    out = pl.pallas_call(
        body,
        out_shape=jax.ShapeDtypeStruct((B, H * D), jnp.float32),
        in_specs=[pl.BlockSpec(memory_space=pltpu.VMEM)] * 5,
        out_specs=pl.BlockSpec(memory_space=pltpu.VMEM),
        scratch_shapes=[
            pltpu.VMEM((B, MSG), jnp.float32),
            pltpu.VMEM((B, MSG), jnp.float32),
            pltpu.SemaphoreType.DMA,
            pltpu.SemaphoreType.DMA,
        ],
        compiler_params=pltpu.CompilerParams(collective_id=0),
    )(Q2, K2, V2, bt, lens2)
    return out.reshape(B, 1, H, D)
